# Initial kernel scaffold; baseline (speedup 1.0000x reference)
import jax
import jax.numpy as jnp
from jax.experimental import pallas as pl


def kernel(msg, index, t, dim_size, Wih, Whh, bih, bhh, Wmlp, bmlp):
    raise NotImplementedError("write your pallas kernel here")


# speedup vs baseline: 5.5481x; 5.5481x over previous
import functools

import jax
import jax.numpy as jnp
from jax import lax
from jax.experimental import pallas as pl
from jax.experimental.pallas import tpu as pltpu
from jax.experimental.pallas import tpu_sc as plsc

N = 16384
D = 256
H = 128
G = 3 * H
OUT = 256
DS = 2048
C_STEPS = 32
XT_ROWS = C_STEPS * DS + 8
TRASH = C_STEPS * DS

NC, NS = 2, 16
NW = NC * NS
ROWS_PER_W = N // NW
KCH = 128


def _gi_body(m_ref, w_ref, o_ref):
    o_ref[...] = jnp.dot(m_ref[...], w_ref[...],
                         preferred_element_type=jnp.float32)


_gi_call = pl.pallas_call(
    _gi_body,
    grid=(N // 2048,),
    in_specs=[pl.BlockSpec((2048, D), lambda i: (i, 0)),
              pl.BlockSpec((D, G), lambda i: (0, 0))],
    out_specs=pl.BlockSpec((2048, G), lambda i: (i, 0)),
    out_shape=jax.ShapeDtypeStruct((N, G), jnp.float32),
)


@functools.cache
def _make_sc_scatter():
    mesh = plsc.VectorSubcoreMesh(core_axis_name="c", subcore_axis_name="s",
                                  num_cores=NC, num_subcores=NS)

    @functools.partial(
        pl.kernel,
        out_type=jax.ShapeDtypeStruct((XT_ROWS, G), jnp.float32),
        mesh=mesh,
        scratch_types=[pltpu.VMEM((KCH,), jnp.int32),
                       pltpu.VMEM((KCH, G), jnp.float32),
                       pltpu.SemaphoreType.DMA],
    )
    def sc_scatter(gi_hbm, dst_hbm, xt_hbm, idx_v, rows_v, sem):
        wid = lax.axis_index("s") * NC + lax.axis_index("c")
        base = wid * ROWS_PER_W
        for c in range(ROWS_PER_W // KCH):
            off = base + c * KCH
            pltpu.sync_copy(dst_hbm.at[pl.ds(off, KCH)], idx_v)
            pltpu.sync_copy(gi_hbm.at[pl.ds(off, KCH), :], rows_v)
            pltpu.async_copy(rows_v, xt_hbm.at[idx_v], sem).wait()

    return sc_scatter


def _rec_body(scal_ref, xt_hbm, counts_ref, h_ref, whht_ref, wmlpt_ref,
              bih_ref, bhh_ref, bmlp_ref, hout_ref, out_ref, xt_buf, sems):
    nsteps = scal_ref[0]
    t0 = scal_ref[1]
    dimsz = scal_ref[2]
    counts = counts_ref[...]
    whht = whht_ref[...]
    bih = bih_ref[...]
    bhh = bhh_ref[...]

    def _copy(tau, slot):
        return pltpu.make_async_copy(
            xt_hbm.at[pl.ds(tau * DS, DS), :], xt_buf.at[slot], sems.at[slot])

    _copy(0, 0).start()

    def step(tau, h):
        slot = tau % 2

        @pl.when(tau + 1 < nsteps)
        def _():
            _copy(tau + 1, (tau + 1) % 2).start()

        _copy(tau, slot).wait()
        xt = xt_buf[slot]
        gh = jnp.dot(h, whht, preferred_element_type=jnp.float32) + bhh
        gi = jnp.where((t0 + tau) < counts, xt, 0.0) + bih
        r = jax.nn.sigmoid(gi[:, 0:H] + gh[:, 0:H])
        z = jax.nn.sigmoid(gi[:, H:2 * H] + gh[:, H:2 * H])
        n = jnp.tanh(gi[:, 2 * H:] + r * gh[:, 2 * H:])
        return (1.0 - z) * n + z * h

    h = lax.fori_loop(0, nsteps, step, h_ref[...])
    hout_ref[...] = h
    o = jnp.dot(h, wmlpt_ref[...], preferred_element_type=jnp.float32) \
        + bmlp_ref[...]
    row = lax.broadcasted_iota(jnp.int32, (DS, 1), 0)
    keep = (counts > 0) & (row < dimsz)
    out_ref[...] = jnp.where(keep, o, 0.0)


_rec_call = pl.pallas_call(
    _rec_body,
    in_specs=[pl.BlockSpec(memory_space=pltpu.SMEM),
              pl.BlockSpec(memory_space=pl.ANY),
              pl.BlockSpec(memory_space=pltpu.VMEM),
              pl.BlockSpec(memory_space=pltpu.VMEM),
              pl.BlockSpec(memory_space=pltpu.VMEM),
              pl.BlockSpec(memory_space=pltpu.VMEM),
              pl.BlockSpec(memory_space=pltpu.VMEM),
              pl.BlockSpec(memory_space=pltpu.VMEM),
              pl.BlockSpec(memory_space=pltpu.VMEM)],
    out_specs=[pl.BlockSpec(memory_space=pltpu.VMEM),
               pl.BlockSpec(memory_space=pltpu.VMEM)],
    out_shape=[jax.ShapeDtypeStruct((DS, H), jnp.float32),
               jax.ShapeDtypeStruct((DS, OUT), jnp.float32)],
    scratch_shapes=[pltpu.VMEM((2, DS, G), jnp.float32),
                    pltpu.SemaphoreType.DMA((2,))],
)


def kernel(msg, index, t, dim_size, Wih, Whh, bih, bhh, Wmlp, bmlp):
    index = index.astype(jnp.int32)
    order = jnp.argsort(t)
    idx_s = index[order]
    key2 = idx_s * N + jnp.arange(N, dtype=jnp.int32)
    perm = jnp.argsort(key2)
    gorder = order[perm]
    idx_g = idx_s[perm]
    counts = jnp.bincount(index, length=DS).astype(jnp.int32)
    offsets = (jnp.cumsum(counts) - counts).astype(jnp.int32)
    pos_g = jnp.arange(N, dtype=jnp.int32) - offsets[idx_g]
    pos = jnp.zeros((N,), jnp.int32).at[gorder].set(pos_g)
    max_len = counts.max()

    gi = _gi_call(msg, Wih.T)

    counts2 = counts.reshape(DS, 1)
    whht = Whh.T
    wmlpt = Wmlp.T
    bih2 = bih.reshape(1, G)
    bhh2 = bhh.reshape(1, G)
    bmlp2 = bmlp.reshape(1, OUT)
    dimsz = jnp.asarray(dim_size, jnp.int32)

    def cond(carry):
        t0, _, _ = carry
        return t0 < max_len

    def body(carry):
        t0, h, _ = carry
        in_chunk = (pos >= t0) & (pos < t0 + C_STEPS)
        dst = jnp.where(in_chunk, (pos - t0) * DS + index, TRASH)
        xt = _make_sc_scatter()(gi, dst)
        nsteps = jnp.minimum(max_len - t0, C_STEPS)
        scal = jnp.stack([nsteps, t0, dimsz]).astype(jnp.int32)
        h, out = _rec_call(scal, xt, counts2, h, whht, wmlpt,
                           bih2, bhh2, bmlp2)
        return (t0 + C_STEPS, h, out)

    carry0 = (jnp.int32(0), jnp.zeros((DS, H), jnp.float32),
              jnp.zeros((DS, OUT), jnp.float32))
    _, _, out = lax.while_loop(cond, body, carry0)
    return out


# speedup vs baseline: 7.1659x; 1.2916x over previous
import functools

import jax
import jax.numpy as jnp
from jax import lax
from jax.experimental import pallas as pl
from jax.experimental.pallas import tpu as pltpu
from jax.experimental.pallas import tpu_sc as plsc

N = 16384
D = 256
H = 128
G = 3 * H
OUT = 256
DS = 2048
C_STEPS = 32
XT_ROWS = C_STEPS * DS + 8
TRASH = C_STEPS * DS

NC, NS = 2, 16
NW = NC * NS
ROWS_PER_W = N // NW
KCH = 128


def _gi_body(m_ref, w_ref, o_ref):
    o_ref[...] = jnp.dot(m_ref[...], w_ref[...],
                         preferred_element_type=jnp.float32)


_gi_call = pl.pallas_call(
    _gi_body,
    grid=(N // 2048,),
    in_specs=[pl.BlockSpec((2048, D), lambda i: (i, 0)),
              pl.BlockSpec((D, G), lambda i: (0, 0))],
    out_specs=pl.BlockSpec((2048, G), lambda i: (i, 0)),
    out_shape=jax.ShapeDtypeStruct((N, G), jnp.float32),
)


@functools.cache
def _make_sc_gs():
    mesh = plsc.VectorSubcoreMesh(core_axis_name="c", subcore_axis_name="s",
                                  num_cores=NC, num_subcores=NS)

    @functools.partial(
        pl.kernel,
        out_type=jax.ShapeDtypeStruct((XT_ROWS, G), jnp.float32),
        mesh=mesh,
        scratch_types=[pltpu.VMEM((KCH,), jnp.int32),
                       pltpu.VMEM((KCH,), jnp.int32),
                       pltpu.VMEM((KCH, G), jnp.float32),
                       pltpu.SemaphoreType.DMA],
    )
    def sc_gs(gi_hbm, src_hbm, dst_hbm, xt_hbm, isrc_v, idst_v, rows_v, sem):
        wid = lax.axis_index("s") * NC + lax.axis_index("c")
        base = wid * ROWS_PER_W
        for c in range(ROWS_PER_W // KCH):
            off = base + c * KCH
            pltpu.sync_copy(src_hbm.at[pl.ds(off, KCH)], isrc_v)
            pltpu.sync_copy(dst_hbm.at[pl.ds(off, KCH)], idst_v)
            pltpu.async_copy(gi_hbm.at[isrc_v], rows_v, sem).wait()
            pltpu.async_copy(rows_v, xt_hbm.at[idst_v], sem).wait()

    return sc_gs


def _rec_body(scal_ref, xt_hbm, counts_ref, h_ref, whht_ref, wmlpt_ref,
              bih_ref, bhh_ref, bmlp_ref, hout_ref, out_ref, xt_buf, sems):
    nsteps = scal_ref[0]
    t0 = scal_ref[1]
    dimsz = scal_ref[2]
    counts = counts_ref[...]
    whht = whht_ref[...]
    bih = bih_ref[...]
    bhh = bhh_ref[...]

    def _copy(tau, slot):
        return pltpu.make_async_copy(
            xt_hbm.at[pl.ds(tau * DS, DS), :], xt_buf.at[slot], sems.at[slot])

    _copy(0, 0).start()

    def step(tau, h):
        slot = tau % 2

        @pl.when(tau + 1 < nsteps)
        def _():
            _copy(tau + 1, (tau + 1) % 2).start()

        _copy(tau, slot).wait()
        xt = xt_buf[slot]
        gh = jnp.dot(h, whht, preferred_element_type=jnp.float32) + bhh
        gi = jnp.where((t0 + tau) < counts, xt, 0.0) + bih
        r = jax.nn.sigmoid(gi[:, 0:H] + gh[:, 0:H])
        z = jax.nn.sigmoid(gi[:, H:2 * H] + gh[:, H:2 * H])
        n = jnp.tanh(gi[:, 2 * H:] + r * gh[:, 2 * H:])
        return (1.0 - z) * n + z * h

    h = lax.fori_loop(0, nsteps, step, h_ref[...])
    hout_ref[...] = h
    o = jnp.dot(h, wmlpt_ref[...], preferred_element_type=jnp.float32) \
        + bmlp_ref[...]
    row = lax.broadcasted_iota(jnp.int32, (DS, 1), 0)
    keep = (counts > 0) & (row < dimsz)
    out_ref[...] = jnp.where(keep, o, 0.0)


_rec_call = pl.pallas_call(
    _rec_body,
    in_specs=[pl.BlockSpec(memory_space=pltpu.SMEM),
              pl.BlockSpec(memory_space=pl.ANY),
              pl.BlockSpec(memory_space=pltpu.VMEM),
              pl.BlockSpec(memory_space=pltpu.VMEM),
              pl.BlockSpec(memory_space=pltpu.VMEM),
              pl.BlockSpec(memory_space=pltpu.VMEM),
              pl.BlockSpec(memory_space=pltpu.VMEM),
              pl.BlockSpec(memory_space=pltpu.VMEM),
              pl.BlockSpec(memory_space=pltpu.VMEM)],
    out_specs=[pl.BlockSpec(memory_space=pltpu.VMEM),
               pl.BlockSpec(memory_space=pltpu.VMEM)],
    out_shape=[jax.ShapeDtypeStruct((DS, H), jnp.float32),
               jax.ShapeDtypeStruct((DS, OUT), jnp.float32)],
    scratch_shapes=[pltpu.VMEM((2, DS, G), jnp.float32),
                    pltpu.SemaphoreType.DMA((2,))],
)


def kernel(msg, index, t, dim_size, Wih, Whh, bih, bhh, Wmlp, bmlp):
    index = index.astype(jnp.int32)
    ar = jnp.arange(N, dtype=jnp.int32)
    _, idx_s, order = lax.sort((t, index, ar), num_keys=1, is_stable=True)
    idx_g, gorder = lax.sort((idx_s, order), num_keys=1, is_stable=True)
    counts = jnp.bincount(index, length=DS).astype(jnp.int32)
    offsets = (jnp.cumsum(counts) - counts).astype(jnp.int32)
    pos_g = ar - offsets[idx_g]
    max_len = counts.max()

    gi = _gi_call(msg, Wih.T)

    counts2 = counts.reshape(DS, 1)
    whht = Whh.T
    wmlpt = Wmlp.T
    bih2 = bih.reshape(1, G)
    bhh2 = bhh.reshape(1, G)
    bmlp2 = bmlp.reshape(1, OUT)
    dimsz = jnp.asarray(dim_size, jnp.int32)

    def cond(carry):
        t0, _, _ = carry
        return t0 < max_len

    def body(carry):
        t0, h, _ = carry
        in_chunk = (pos_g >= t0) & (pos_g < t0 + C_STEPS)
        dst = jnp.where(in_chunk, (pos_g - t0) * DS + idx_g, TRASH)
        xt = _make_sc_gs()(gi, gorder, dst)
        nsteps = jnp.minimum(max_len - t0, C_STEPS)
        scal = jnp.stack([nsteps, t0, dimsz]).astype(jnp.int32)
        h, out = _rec_call(scal, xt, counts2, h, whht, wmlpt,
                           bih2, bhh2, bmlp2)
        return (t0 + C_STEPS, h, out)

    carry0 = (jnp.int32(0), jnp.zeros((DS, H), jnp.float32),
              jnp.zeros((DS, OUT), jnp.float32))
    _, _, out = lax.while_loop(cond, body, carry0)
    return out


# speedup vs baseline: 10.4998x; 1.4652x over previous
import functools

import jax
import jax.numpy as jnp
from jax import lax
from jax.experimental import pallas as pl
from jax.experimental.pallas import tpu as pltpu
from jax.experimental.pallas import tpu_sc as plsc

N = 16384
D = 256
H = 128
G = 3 * H
OUT = 256
DS = 2048
C_STEPS = 32
XT_ROWS = C_STEPS * DS + 8
TRASH = C_STEPS * DS

NC, NS = 2, 16
NW = NC * NS
ROWS_PER_W = N // NW
KCH = 128


def _gi_body(m_ref, w_ref, o_ref):
    o_ref[...] = jnp.dot(m_ref[...], w_ref[...],
                         preferred_element_type=jnp.float32)


_gi_call = pl.pallas_call(
    _gi_body,
    grid=(N // 2048,),
    in_specs=[pl.BlockSpec((2048, D), lambda i: (i, 0)),
              pl.BlockSpec((D, G), lambda i: (0, 0))],
    out_specs=pl.BlockSpec((2048, G), lambda i: (i, 0)),
    out_shape=jax.ShapeDtypeStruct((N, G), jnp.float32),
)


@functools.cache
def _make_sc_gs():
    mesh = plsc.VectorSubcoreMesh(core_axis_name="c", subcore_axis_name="s",
                                  num_cores=NC, num_subcores=NS)

    @functools.partial(
        pl.kernel,
        out_type=jax.ShapeDtypeStruct((XT_ROWS, G), jnp.float32),
        mesh=mesh,
        scratch_types=[pltpu.VMEM((KCH,), jnp.int32),
                       pltpu.VMEM((KCH,), jnp.int32),
                       pltpu.VMEM((KCH, G), jnp.float32),
                       pltpu.SemaphoreType.DMA],
    )
    def sc_gs(gi_hbm, src_hbm, dst_hbm, xt_hbm, isrc_v, idst_v, rows_v, sem):
        wid = lax.axis_index("s") * NC + lax.axis_index("c")
        base = wid * ROWS_PER_W
        for c in range(ROWS_PER_W // KCH):
            off = base + c * KCH
            pltpu.sync_copy(src_hbm.at[pl.ds(off, KCH)], isrc_v)
            pltpu.sync_copy(dst_hbm.at[pl.ds(off, KCH)], idst_v)
            pltpu.async_copy(gi_hbm.at[isrc_v], rows_v, sem).wait()
            pltpu.async_copy(rows_v, xt_hbm.at[idst_v], sem).wait()

    return sc_gs


def _rec_body(scal_ref, xt_hbm, counts_ref, h_ref, whht_ref, wmlpt_ref,
              bih_ref, bhh_ref, bmlp_ref, hout_ref, out_ref, xt_buf, sems):
    nsteps = scal_ref[0]
    t0 = scal_ref[1]
    dimsz = scal_ref[2]
    counts = counts_ref[...]
    whht = whht_ref[...]
    bih = bih_ref[...]
    bhh = bhh_ref[...]

    def _copy(tau, slot):
        return pltpu.make_async_copy(
            xt_hbm.at[pl.ds(tau * DS, DS), :], xt_buf.at[slot], sems.at[slot])

    _copy(0, 0).start()

    def step(tau, h):
        slot = tau % 2

        @pl.when(tau + 1 < nsteps)
        def _():
            _copy(tau + 1, (tau + 1) % 2).start()

        _copy(tau, slot).wait()
        xt = xt_buf[slot]
        gh = jnp.dot(h, whht, preferred_element_type=jnp.float32) + bhh
        gi = jnp.where((t0 + tau) < counts, xt, 0.0) + bih
        r = jax.nn.sigmoid(gi[:, 0:H] + gh[:, 0:H])
        z = jax.nn.sigmoid(gi[:, H:2 * H] + gh[:, H:2 * H])
        n = jnp.tanh(gi[:, 2 * H:] + r * gh[:, 2 * H:])
        return (1.0 - z) * n + z * h

    h = lax.fori_loop(0, nsteps, step, h_ref[...])
    hout_ref[...] = h
    o = jnp.dot(h, wmlpt_ref[...], preferred_element_type=jnp.float32) \
        + bmlp_ref[...]
    row = lax.broadcasted_iota(jnp.int32, (DS, 1), 0)
    keep = (counts > 0) & (row < dimsz)
    out_ref[...] = jnp.where(keep, o, 0.0)


_rec_call = pl.pallas_call(
    _rec_body,
    in_specs=[pl.BlockSpec(memory_space=pltpu.SMEM),
              pl.BlockSpec(memory_space=pl.ANY),
              pl.BlockSpec(memory_space=pltpu.VMEM),
              pl.BlockSpec(memory_space=pltpu.VMEM),
              pl.BlockSpec(memory_space=pltpu.VMEM),
              pl.BlockSpec(memory_space=pltpu.VMEM),
              pl.BlockSpec(memory_space=pltpu.VMEM),
              pl.BlockSpec(memory_space=pltpu.VMEM),
              pl.BlockSpec(memory_space=pltpu.VMEM)],
    out_specs=[pl.BlockSpec(memory_space=pltpu.VMEM),
               pl.BlockSpec(memory_space=pltpu.VMEM)],
    out_shape=[jax.ShapeDtypeStruct((DS, H), jnp.float32),
               jax.ShapeDtypeStruct((DS, OUT), jnp.float32)],
    scratch_shapes=[pltpu.VMEM((2, DS, G), jnp.float32),
                    pltpu.SemaphoreType.DMA((2,))],
)


def kernel(msg, index, t, dim_size, Wih, Whh, bih, bhh, Wmlp, bmlp):
    index = index.astype(jnp.int32)
    ar = jnp.arange(N, dtype=jnp.int32)
    idx_g, _, gorder = lax.sort((index, t, ar), num_keys=2, is_stable=True)
    flag = jnp.concatenate([jnp.ones((1,), jnp.bool_), idx_g[1:] != idx_g[:-1]])
    starts = lax.cummax(jnp.where(flag, ar, 0))
    pos_g = ar - starts
    counts = jnp.zeros((DS,), jnp.int32).at[idx_g].max(pos_g + 1, mode="drop")
    max_len = pos_g.max() + 1

    gi = _gi_call(msg, Wih.T)

    counts2 = counts.reshape(DS, 1)
    whht = Whh.T
    wmlpt = Wmlp.T
    bih2 = bih.reshape(1, G)
    bhh2 = bhh.reshape(1, G)
    bmlp2 = bmlp.reshape(1, OUT)
    dimsz = jnp.asarray(dim_size, jnp.int32)

    def cond(carry):
        t0, _, _ = carry
        return t0 < max_len

    def body(carry):
        t0, h, _ = carry
        in_chunk = (pos_g >= t0) & (pos_g < t0 + C_STEPS)
        dst = jnp.where(in_chunk, (pos_g - t0) * DS + idx_g, TRASH)
        xt = _make_sc_gs()(gi, gorder, dst)
        nsteps = jnp.minimum(max_len - t0, C_STEPS)
        scal = jnp.stack([nsteps, t0, dimsz]).astype(jnp.int32)
        h, out = _rec_call(scal, xt, counts2, h, whht, wmlpt,
                           bih2, bhh2, bmlp2)
        return (t0 + C_STEPS, h, out)

    carry0 = (jnp.int32(0), jnp.zeros((DS, H), jnp.float32),
              jnp.zeros((DS, OUT), jnp.float32))
    _, _, out = lax.while_loop(cond, body, carry0)
    return out


# speedup vs baseline: 15.0801x; 1.4362x over previous
import functools

import jax
import jax.numpy as jnp
from jax import lax
from jax.experimental import pallas as pl
from jax.experimental.pallas import tpu as pltpu
from jax.experimental.pallas import tpu_sc as plsc

N = 16384
D = 256
H = 128
G = 3 * H
OUT = 256
DS = 2048
C_STEPS = 32
XT_ROWS = C_STEPS * DS + 8
TRASH = C_STEPS * DS

NC, NS = 2, 16
NW = NC * NS
ROWS_PER_W = N // NW
KCH = 128


def _gi_body(m_ref, w_ref, o_ref):
    o_ref[...] = jnp.dot(m_ref[...], w_ref[...],
                         preferred_element_type=jnp.float32)


_gi_call = pl.pallas_call(
    _gi_body,
    grid=(N // 2048,),
    in_specs=[pl.BlockSpec((2048, D), lambda i: (i, 0)),
              pl.BlockSpec((D, G), lambda i: (0, 0))],
    out_specs=pl.BlockSpec((2048, G), lambda i: (i, 0)),
    out_shape=jax.ShapeDtypeStruct((N, G), jnp.float32),
)


@functools.cache
def _make_sc_gs():
    mesh = plsc.VectorSubcoreMesh(core_axis_name="c", subcore_axis_name="s",
                                  num_cores=NC, num_subcores=NS)

    @functools.partial(
        pl.kernel,
        out_type=jax.ShapeDtypeStruct((XT_ROWS, G), jnp.float32),
        mesh=mesh,
        scratch_types=[pltpu.VMEM((KCH,), jnp.int32),
                       pltpu.VMEM((KCH,), jnp.int32),
                       pltpu.VMEM((KCH, G), jnp.float32),
                       pltpu.SemaphoreType.DMA],
    )
    def sc_gs(gi_hbm, src_hbm, dst_hbm, xt_hbm, isrc_v, idst_v, rows_v, sem):
        wid = lax.axis_index("s") * NC + lax.axis_index("c")
        base = wid * ROWS_PER_W
        for c in range(ROWS_PER_W // KCH):
            off = base + c * KCH
            pltpu.sync_copy(src_hbm.at[pl.ds(off, KCH)], isrc_v)
            pltpu.sync_copy(dst_hbm.at[pl.ds(off, KCH)], idst_v)
            pltpu.async_copy(gi_hbm.at[isrc_v], rows_v, sem).wait()
            pltpu.async_copy(rows_v, xt_hbm.at[idst_v], sem).wait()

    return sc_gs


def _rec_body(scal_ref, xt_hbm, counts_ref, h_ref, whht_ref, wmlpt_ref,
              bih_ref, bhh_ref, bmlp_ref, hout_ref, out_ref, xt_buf, sems):
    nsteps = scal_ref[0]
    t0 = scal_ref[1]
    dimsz = scal_ref[2]
    counts = counts_ref[...]
    whht = whht_ref[...]
    bih = bih_ref[...]
    bhh = bhh_ref[...]

    def _copy(tau, slot):
        return pltpu.make_async_copy(
            xt_hbm.at[pl.ds(tau * DS, DS), :], xt_buf.at[slot], sems.at[slot])

    _copy(0, 0).start()

    def step(tau, h):
        slot = tau % 2

        @pl.when(tau + 1 < nsteps)
        def _():
            _copy(tau + 1, (tau + 1) % 2).start()

        _copy(tau, slot).wait()
        xt = xt_buf[slot]
        gh = jnp.dot(h, whht, preferred_element_type=jnp.float32) + bhh
        gi = jnp.where((t0 + tau) < counts, xt, 0.0) + bih
        r = jax.nn.sigmoid(gi[:, 0:H] + gh[:, 0:H])
        z = jax.nn.sigmoid(gi[:, H:2 * H] + gh[:, H:2 * H])
        n = jnp.tanh(gi[:, 2 * H:] + r * gh[:, 2 * H:])
        return (1.0 - z) * n + z * h

    h = lax.fori_loop(0, nsteps, step, h_ref[...])
    hout_ref[...] = h
    o = jnp.dot(h, wmlpt_ref[...], preferred_element_type=jnp.float32) \
        + bmlp_ref[...]
    row = lax.broadcasted_iota(jnp.int32, (DS, 1), 0)
    keep = (counts > 0) & (row < dimsz)
    out_ref[...] = jnp.where(keep, o, 0.0)


_rec_call = pl.pallas_call(
    _rec_body,
    in_specs=[pl.BlockSpec(memory_space=pltpu.SMEM),
              pl.BlockSpec(memory_space=pl.ANY),
              pl.BlockSpec(memory_space=pltpu.VMEM),
              pl.BlockSpec(memory_space=pltpu.VMEM),
              pl.BlockSpec(memory_space=pltpu.VMEM),
              pl.BlockSpec(memory_space=pltpu.VMEM),
              pl.BlockSpec(memory_space=pltpu.VMEM),
              pl.BlockSpec(memory_space=pltpu.VMEM),
              pl.BlockSpec(memory_space=pltpu.VMEM)],
    out_specs=[pl.BlockSpec(memory_space=pltpu.VMEM),
               pl.BlockSpec(memory_space=pltpu.VMEM)],
    out_shape=[jax.ShapeDtypeStruct((DS, H), jnp.float32),
               jax.ShapeDtypeStruct((DS, OUT), jnp.float32)],
    scratch_shapes=[pltpu.VMEM((2, DS, G), jnp.float32),
                    pltpu.SemaphoreType.DMA((2,))],
)


def kernel(msg, index, t, dim_size, Wih, Whh, bih, bhh, Wmlp, bmlp):
    index = index.astype(jnp.int32)
    ar = jnp.arange(N, dtype=jnp.int32)
    idx_g, _, gorder = lax.sort((index, t, ar), num_keys=2, is_stable=True)
    flag = jnp.concatenate([jnp.ones((1,), jnp.bool_), idx_g[1:] != idx_g[:-1]])
    starts = lax.cummax(jnp.where(flag, ar, 0))
    pos_g = ar - starts
    counts = jnp.bincount(index, length=DS).astype(jnp.int32)
    max_len = pos_g.max() + 1

    gi = _gi_call(msg, Wih.T)

    counts2 = counts.reshape(DS, 1)
    whht = Whh.T
    wmlpt = Wmlp.T
    bih2 = bih.reshape(1, G)
    bhh2 = bhh.reshape(1, G)
    bmlp2 = bmlp.reshape(1, OUT)
    dimsz = jnp.asarray(dim_size, jnp.int32)

    def cond(carry):
        t0, _, _ = carry
        return t0 < max_len

    def body(carry):
        t0, h, _ = carry
        in_chunk = (pos_g >= t0) & (pos_g < t0 + C_STEPS)
        dst = jnp.where(in_chunk, (pos_g - t0) * DS + idx_g, TRASH)
        xt = _make_sc_gs()(gi, gorder, dst)
        nsteps = jnp.minimum(max_len - t0, C_STEPS)
        scal = jnp.stack([nsteps, t0, dimsz]).astype(jnp.int32)
        h, out = _rec_call(scal, xt, counts2, h, whht, wmlpt,
                           bih2, bhh2, bmlp2)
        return (t0 + C_STEPS, h, out)

    carry0 = (jnp.int32(0), jnp.zeros((DS, H), jnp.float32),
              jnp.zeros((DS, OUT), jnp.float32))
    _, _, out = lax.while_loop(cond, body, carry0)
    return out


# speedup vs baseline: 15.6735x; 1.0394x over previous
import functools

import jax
import jax.numpy as jnp
from jax import lax
from jax.experimental import pallas as pl
from jax.experimental.pallas import tpu as pltpu
from jax.experimental.pallas import tpu_sc as plsc

N = 16384
D = 256
H = 128
G = 3 * H
OUT = 256
DS = 2048
C_STEPS = 32
XT_ROWS = C_STEPS * DS + 8
TRASH = C_STEPS * DS

NC, NS = 2, 16
NW = NC * NS
ROWS_PER_W = N // NW
KCH = 128


def _gi_body(m_ref, w_ref, o_ref):
    o_ref[...] = jnp.dot(m_ref[...], w_ref[...],
                         preferred_element_type=jnp.float32)


_gi_call = pl.pallas_call(
    _gi_body,
    grid=(N // 2048,),
    in_specs=[pl.BlockSpec((2048, D), lambda i: (i, 0)),
              pl.BlockSpec((D, G), lambda i: (0, 0))],
    out_specs=pl.BlockSpec((2048, G), lambda i: (i, 0)),
    out_shape=jax.ShapeDtypeStruct((N, G), jnp.float32),
)


NCH = ROWS_PER_W // KCH


@functools.cache
def _make_sc_gs():
    mesh = plsc.VectorSubcoreMesh(core_axis_name="c", subcore_axis_name="s",
                                  num_cores=NC, num_subcores=NS)

    @functools.partial(
        pl.kernel,
        out_type=jax.ShapeDtypeStruct((XT_ROWS, G), jnp.float32),
        mesh=mesh,
        scratch_types=[pltpu.VMEM((NCH, KCH), jnp.int32),
                       pltpu.VMEM((NCH, KCH), jnp.int32),
                       pltpu.VMEM((2, KCH, G), jnp.float32),
                       pltpu.SemaphoreType.DMA,
                       pltpu.SemaphoreType.DMA,
                       pltpu.SemaphoreType.DMA,
                       pltpu.SemaphoreType.DMA],
    )
    def sc_gs(gi_hbm, src_hbm, dst_hbm, xt_hbm, srcm, dstm, rows,
              sg0, sg1, ss0, ss1):
        wid = lax.axis_index("s") * NC + lax.axis_index("c")
        sg = (sg0, sg1)
        ss = (ss0, ss1)
        pltpu.sync_copy(src_hbm.at[wid], srcm)
        pltpu.sync_copy(dst_hbm.at[wid], dstm)
        gathers = [None] * NCH
        scatters = [None] * NCH
        gathers[0] = pltpu.async_copy(gi_hbm.at[srcm.at[0]], rows.at[0],
                                      sg[0])
        for c in range(NCH):
            b = c % 2
            nb = (c + 1) % 2
            if c + 1 < NCH:
                if c >= 1:
                    scatters[c - 1].wait()
                gathers[c + 1] = pltpu.async_copy(
                    gi_hbm.at[srcm.at[c + 1]], rows.at[nb], sg[nb])
            gathers[c].wait()
            scatters[c] = pltpu.async_copy(rows.at[b], xt_hbm.at[dstm.at[c]],
                                           ss[b])
        scatters[NCH - 2].wait()
        scatters[NCH - 1].wait()

    return sc_gs


def _rec_body(scal_ref, xt_hbm, counts_ref, h_ref, whht_ref, wmlpt_ref,
              bih_ref, bhh_ref, bmlp_ref, hout_ref, out_ref, xt_buf, sems):
    nsteps = scal_ref[0]
    t0 = scal_ref[1]
    dimsz = scal_ref[2]
    counts = counts_ref[...]
    whht = whht_ref[...]
    bih = bih_ref[...]
    bhh = bhh_ref[...]

    def _copy(tau, slot):
        return pltpu.make_async_copy(
            xt_hbm.at[pl.ds(tau * DS, DS), :], xt_buf.at[slot], sems.at[slot])

    _copy(0, 0).start()

    def step(tau, h):
        slot = tau % 2

        @pl.when(tau + 1 < nsteps)
        def _():
            _copy(tau + 1, (tau + 1) % 2).start()

        _copy(tau, slot).wait()
        xt = xt_buf[slot]
        gh = jnp.dot(h, whht, preferred_element_type=jnp.float32) + bhh
        gi = jnp.where((t0 + tau) < counts, xt, 0.0) + bih
        r = jax.nn.sigmoid(gi[:, 0:H] + gh[:, 0:H])
        z = jax.nn.sigmoid(gi[:, H:2 * H] + gh[:, H:2 * H])
        n = jnp.tanh(gi[:, 2 * H:] + r * gh[:, 2 * H:])
        return (1.0 - z) * n + z * h

    h = lax.fori_loop(0, nsteps, step, h_ref[...])
    hout_ref[...] = h
    o = jnp.dot(h, wmlpt_ref[...], preferred_element_type=jnp.float32) \
        + bmlp_ref[...]
    row = lax.broadcasted_iota(jnp.int32, (DS, 1), 0)
    keep = (counts > 0) & (row < dimsz)
    out_ref[...] = jnp.where(keep, o, 0.0)


_rec_call = pl.pallas_call(
    _rec_body,
    in_specs=[pl.BlockSpec(memory_space=pltpu.SMEM),
              pl.BlockSpec(memory_space=pl.ANY),
              pl.BlockSpec(memory_space=pltpu.VMEM),
              pl.BlockSpec(memory_space=pltpu.VMEM),
              pl.BlockSpec(memory_space=pltpu.VMEM),
              pl.BlockSpec(memory_space=pltpu.VMEM),
              pl.BlockSpec(memory_space=pltpu.VMEM),
              pl.BlockSpec(memory_space=pltpu.VMEM),
              pl.BlockSpec(memory_space=pltpu.VMEM)],
    out_specs=[pl.BlockSpec(memory_space=pltpu.VMEM),
               pl.BlockSpec(memory_space=pltpu.VMEM)],
    out_shape=[jax.ShapeDtypeStruct((DS, H), jnp.float32),
               jax.ShapeDtypeStruct((DS, OUT), jnp.float32)],
    scratch_shapes=[pltpu.VMEM((2, DS, G), jnp.float32),
                    pltpu.SemaphoreType.DMA((2,))],
)


def kernel(msg, index, t, dim_size, Wih, Whh, bih, bhh, Wmlp, bmlp):
    index = index.astype(jnp.int32)
    ar = jnp.arange(N, dtype=jnp.int32)
    idx_g, _, gorder = lax.sort((index, t, ar), num_keys=2, is_stable=True)
    flag = jnp.concatenate([jnp.ones((1,), jnp.bool_), idx_g[1:] != idx_g[:-1]])
    starts = lax.cummax(jnp.where(flag, ar, 0))
    pos_g = ar - starts
    counts = jnp.bincount(index, length=DS).astype(jnp.int32)
    max_len = pos_g.max() + 1

    gi = _gi_call(msg, Wih.T)

    counts2 = counts.reshape(DS, 1)
    whht = Whh.T
    wmlpt = Wmlp.T
    bih2 = bih.reshape(1, G)
    bhh2 = bhh.reshape(1, G)
    bmlp2 = bmlp.reshape(1, OUT)
    dimsz = jnp.asarray(dim_size, jnp.int32)

    def cond(carry):
        t0, _, _ = carry
        return t0 < max_len

    def body(carry):
        t0, h, _ = carry
        in_chunk = (pos_g >= t0) & (pos_g < t0 + C_STEPS)
        dst = jnp.where(in_chunk, (pos_g - t0) * DS + idx_g, TRASH)
        xt = _make_sc_gs()(gi, gorder.reshape(NW, NCH, KCH),
                           dst.reshape(NW, NCH, KCH))
        nsteps = jnp.minimum(max_len - t0, C_STEPS)
        scal = jnp.stack([nsteps, t0, dimsz]).astype(jnp.int32)
        h, out = _rec_call(scal, xt, counts2, h, whht, wmlpt,
                           bih2, bhh2, bmlp2)
        return (t0 + C_STEPS, h, out)

    carry0 = (jnp.int32(0), jnp.zeros((DS, H), jnp.float32),
              jnp.zeros((DS, OUT), jnp.float32))
    _, _, out = lax.while_loop(cond, body, carry0)
    return out


# speedup vs baseline: 15.7896x; 1.0074x over previous
import functools

import jax
import jax.numpy as jnp
from jax import lax
from jax.experimental import pallas as pl
from jax.experimental.pallas import tpu as pltpu
from jax.experimental.pallas import tpu_sc as plsc

N = 16384
D = 256
H = 128
G = 3 * H
OUT = 256
DS = 2048
C_STEPS = 32
XT_ROWS = C_STEPS * DS + 8
TRASH = C_STEPS * DS

NC, NS = 2, 16
NW = NC * NS
ROWS_PER_W = N // NW
KCH = 128


def _gi_body(m_ref, w_ref, b_ref, o_ref):
    o_ref[...] = jnp.dot(m_ref[...], w_ref[...],
                         preferred_element_type=jnp.float32) + b_ref[...]


_gi_call = pl.pallas_call(
    _gi_body,
    grid=(N // 2048,),
    in_specs=[pl.BlockSpec((2048, D), lambda i: (i, 0)),
              pl.BlockSpec((D, G), lambda i: (0, 0)),
              pl.BlockSpec((1, G), lambda i: (0, 0))],
    out_specs=pl.BlockSpec((2048, G), lambda i: (i, 0)),
    out_shape=jax.ShapeDtypeStruct((N, G), jnp.float32),
)


NCH = ROWS_PER_W // KCH


@functools.cache
def _make_sc_gs():
    mesh = plsc.VectorSubcoreMesh(core_axis_name="c", subcore_axis_name="s",
                                  num_cores=NC, num_subcores=NS)

    @functools.partial(
        pl.kernel,
        out_type=jax.ShapeDtypeStruct((XT_ROWS, G), jnp.float32),
        mesh=mesh,
        scratch_types=[pltpu.VMEM((NCH, KCH), jnp.int32),
                       pltpu.VMEM((NCH, KCH), jnp.int32),
                       pltpu.VMEM((2, KCH, G), jnp.float32),
                       pltpu.SemaphoreType.DMA,
                       pltpu.SemaphoreType.DMA,
                       pltpu.SemaphoreType.DMA,
                       pltpu.SemaphoreType.DMA],
    )
    def sc_gs(gi_hbm, src_hbm, dst_hbm, xt_hbm, srcm, dstm, rows,
              sg0, sg1, ss0, ss1):
        wid = lax.axis_index("s") * NC + lax.axis_index("c")
        sg = (sg0, sg1)
        ss = (ss0, ss1)
        pltpu.sync_copy(src_hbm.at[wid], srcm)
        pltpu.sync_copy(dst_hbm.at[wid], dstm)
        gathers = [None] * NCH
        scatters = [None] * NCH
        gathers[0] = pltpu.async_copy(gi_hbm.at[srcm.at[0]], rows.at[0],
                                      sg[0])
        for c in range(NCH):
            b = c % 2
            nb = (c + 1) % 2
            if c + 1 < NCH:
                if c >= 1:
                    scatters[c - 1].wait()
                gathers[c + 1] = pltpu.async_copy(
                    gi_hbm.at[srcm.at[c + 1]], rows.at[nb], sg[nb])
            gathers[c].wait()
            scatters[c] = pltpu.async_copy(rows.at[b], xt_hbm.at[dstm.at[c]],
                                           ss[b])
        scatters[NCH - 2].wait()
        scatters[NCH - 1].wait()

    return sc_gs


def _rec_body(scal_ref, xt_hbm, counts_ref, h_ref, whht_ref, wmlpt_ref,
              bih_ref, bhh_ref, bmlp_ref, hout_ref, out_ref, xt_buf, sems):
    nsteps = scal_ref[0]
    t0 = scal_ref[1]
    dimsz = scal_ref[2]
    counts = counts_ref[...]
    whht = whht_ref[...]
    bih = bih_ref[...]
    bhh = bhh_ref[...]

    def _copy(tau, slot):
        return pltpu.make_async_copy(
            xt_hbm.at[pl.ds(tau * DS, DS), :], xt_buf.at[slot], sems.at[slot])

    _copy(0, 0).start()

    def step(tau, h):
        slot = tau % 2

        @pl.when(tau + 1 < nsteps)
        def _():
            _copy(tau + 1, (tau + 1) % 2).start()

        _copy(tau, slot).wait()
        xt = xt_buf[slot]
        gh = jnp.dot(h, whht, preferred_element_type=jnp.float32) + bhh
        gi = jnp.where((t0 + tau) < counts, xt, bih)
        r = jax.nn.sigmoid(gi[:, 0:H] + gh[:, 0:H])
        z = jax.nn.sigmoid(gi[:, H:2 * H] + gh[:, H:2 * H])
        n = jnp.tanh(gi[:, 2 * H:] + r * gh[:, 2 * H:])
        return (1.0 - z) * n + z * h

    h = lax.fori_loop(0, nsteps, step, h_ref[...])
    hout_ref[...] = h
    o = jnp.dot(h, wmlpt_ref[...], preferred_element_type=jnp.float32) \
        + bmlp_ref[...]
    row = lax.broadcasted_iota(jnp.int32, (DS, 1), 0)
    keep = (counts > 0) & (row < dimsz)
    out_ref[...] = jnp.where(keep, o, 0.0)


_rec_call = pl.pallas_call(
    _rec_body,
    in_specs=[pl.BlockSpec(memory_space=pltpu.SMEM),
              pl.BlockSpec(memory_space=pl.ANY),
              pl.BlockSpec(memory_space=pltpu.VMEM),
              pl.BlockSpec(memory_space=pltpu.VMEM),
              pl.BlockSpec(memory_space=pltpu.VMEM),
              pl.BlockSpec(memory_space=pltpu.VMEM),
              pl.BlockSpec(memory_space=pltpu.VMEM),
              pl.BlockSpec(memory_space=pltpu.VMEM),
              pl.BlockSpec(memory_space=pltpu.VMEM)],
    out_specs=[pl.BlockSpec(memory_space=pltpu.VMEM),
               pl.BlockSpec(memory_space=pltpu.VMEM)],
    out_shape=[jax.ShapeDtypeStruct((DS, H), jnp.float32),
               jax.ShapeDtypeStruct((DS, OUT), jnp.float32)],
    scratch_shapes=[pltpu.VMEM((2, DS, G), jnp.float32),
                    pltpu.SemaphoreType.DMA((2,))],
)


def kernel(msg, index, t, dim_size, Wih, Whh, bih, bhh, Wmlp, bmlp):
    index = index.astype(jnp.int32)
    ar = jnp.arange(N, dtype=jnp.int32)
    idx_g, _, gorder = lax.sort((index, t, ar), num_keys=2, is_stable=True)
    flag = jnp.concatenate([jnp.ones((1,), jnp.bool_), idx_g[1:] != idx_g[:-1]])
    starts = lax.cummax(jnp.where(flag, ar, 0))
    pos_g = ar - starts
    counts = jnp.bincount(index, length=DS).astype(jnp.int32)
    max_len = pos_g.max() + 1

    gi = _gi_call(msg, Wih.T, bih.reshape(1, G))

    counts2 = counts.reshape(DS, 1)
    whht = Whh.T
    wmlpt = Wmlp.T
    bih2 = bih.reshape(1, G)
    bhh2 = bhh.reshape(1, G)
    bmlp2 = bmlp.reshape(1, OUT)
    dimsz = jnp.asarray(dim_size, jnp.int32)

    def cond(carry):
        t0, _, _ = carry
        return t0 < max_len

    def body(carry):
        t0, h, _ = carry
        in_chunk = (pos_g >= t0) & (pos_g < t0 + C_STEPS)
        dst = jnp.where(in_chunk, (pos_g - t0) * DS + idx_g, TRASH)
        xt = _make_sc_gs()(gi, gorder.reshape(NW, NCH, KCH),
                           dst.reshape(NW, NCH, KCH))
        nsteps = jnp.minimum(max_len - t0, C_STEPS)
        scal = jnp.stack([nsteps, t0, dimsz]).astype(jnp.int32)
        h, out = _rec_call(scal, xt, counts2, h, whht, wmlpt,
                           bih2, bhh2, bmlp2)
        return (t0 + C_STEPS, h, out)

    carry0 = (jnp.int32(0), jnp.zeros((DS, H), jnp.float32),
              jnp.zeros((DS, OUT), jnp.float32))
    _, _, out = lax.while_loop(cond, body, carry0)
    return out
